# Initial kernel scaffold; baseline (speedup 1.0000x reference)
import jax
import jax.numpy as jnp
from jax.experimental import pallas as pl


def kernel(node_features_batch, graph_src_nodes_batch, graph_dst_nodes_batch, graph_edge_types_batch, W_rel, W_loop, b_gcn, Wq, bq, Wk, bk, Wv, bv, Wo, bo, W_mlp, b_mlp):
    raise NotImplementedError("write your pallas kernel here")


# speedup vs baseline: 5.3360x; 5.3360x over previous
import jax
import jax.numpy as jnp
from jax import lax
from jax.experimental import pallas as pl
from jax.experimental.pallas import tpu as pltpu


def _gcn_attn_body(x_ref, src_ref, dst_ref, et_ref, wrel_ref, wloop_ref,
                   bgcn_ref, wq_ref, bq_ref, wk_ref, bk_ref, wv_ref, bv_ref,
                   wo_ref, bo_ref, wmlp_ref, bmlp_ref, out_ref, u_scr, *,
                   n_rel, n_heads):
    max_len = x_ref.shape[1]
    e = src_ref.shape[2]
    d = x_ref.shape[2]
    dh = d // n_heads
    scale = 1.0 / (dh ** 0.5)

    x = x_ref[0]
    src = src_ref[0, 0, :]
    dst = dst_ref[0, 0, :]
    et = et_ref[0, 0, :]

    col = lax.broadcasted_iota(jnp.int32, (e, max_len), 1)
    psrc = (col == src[:, None]).astype(jnp.float32)
    xe = jnp.dot(psrc, x, preferred_element_type=jnp.float32)

    u_scr[...] = jnp.zeros((e, d), jnp.float32)

    def rel_step(r, carry):
        m = (et == r).astype(jnp.float32)
        w = wrel_ref[r]
        u_scr[...] += jnp.dot(xe * m[:, None], w,
                              preferred_element_type=jnp.float32)
        return carry

    lax.fori_loop(0, n_rel, rel_step, 0)
    u = u_scr[...]

    pdst = (col == dst[:, None]).astype(jnp.float32)
    agg = lax.dot_general(pdst, u, (((0,), (0,)), ((), ())),
                          preferred_element_type=jnp.float32)

    h = (agg + jnp.dot(x, wloop_ref[...], preferred_element_type=jnp.float32)
         + bgcn_ref[...])
    q = jnp.dot(h, wq_ref[...], preferred_element_type=jnp.float32) + bq_ref[...]
    k = jnp.dot(h, wk_ref[...], preferred_element_type=jnp.float32) + bk_ref[...]
    v = jnp.dot(h, wv_ref[...], preferred_element_type=jnp.float32) + bv_ref[...]

    pieces = []
    for hd in range(n_heads):
        sl = slice(hd * dh, (hd + 1) * dh)
        qh = q[:, sl]
        kh = k[:, sl]
        vh = v[:, sl]
        s = lax.dot_general(qh, kh, (((1,), (1,)), ((), ())),
                            preferred_element_type=jnp.float32) * scale
        s = s - jnp.max(s, axis=1, keepdims=True)
        p = jnp.exp(s)
        p = p / jnp.sum(p, axis=1, keepdims=True)
        ctxh = jnp.dot(p, vh, preferred_element_type=jnp.float32)
        pieces.append(jnp.sum(ctxh, axis=0, keepdims=True))
    mc = jnp.concatenate(pieces, axis=1) * (1.0 / max_len)
    f = jnp.dot(mc, wo_ref[...], preferred_element_type=jnp.float32) + bo_ref[...]
    out_ref[0] = (jnp.dot(f, wmlp_ref[...],
                          preferred_element_type=jnp.float32) + bmlp_ref[...])


def kernel(node_features_batch, graph_src_nodes_batch, graph_dst_nodes_batch,
           graph_edge_types_batch, W_rel, W_loop, b_gcn, Wq, bq, Wk, bk, Wv,
           bv, Wo, bo, W_mlp, b_mlp):
    b, max_len, d = node_features_batch.shape
    e = graph_src_nodes_batch.shape[1]
    n_rel = W_rel.shape[0]
    n_heads = 8

    src3 = graph_src_nodes_batch.astype(jnp.int32).reshape(b, 1, e)
    dst3 = graph_dst_nodes_batch.astype(jnp.int32).reshape(b, 1, e)
    et3 = graph_edge_types_batch.astype(jnp.int32).reshape(b, 1, e)
    row = lambda v: v.reshape(1, d)

    import functools
    body = functools.partial(_gcn_attn_body, n_rel=n_rel, n_heads=n_heads)

    batch_spec = pl.BlockSpec((1, max_len, d), lambda i: (i, 0, 0))
    idx_spec = pl.BlockSpec((1, 1, e), lambda i: (i, 0, 0))
    wrel_spec = pl.BlockSpec((n_rel, d, d), lambda i: (0, 0, 0))
    mat_spec = pl.BlockSpec((d, d), lambda i: (0, 0))
    vec_spec = pl.BlockSpec((1, d), lambda i: (0, 0))

    out = pl.pallas_call(
        body,
        grid=(b,),
        in_specs=[batch_spec, idx_spec, idx_spec, idx_spec, wrel_spec,
                  mat_spec, vec_spec, mat_spec, vec_spec, mat_spec, vec_spec,
                  mat_spec, vec_spec, mat_spec, vec_spec, mat_spec, vec_spec],
        out_specs=pl.BlockSpec((1, 1, d), lambda i: (i, 0, 0)),
        out_shape=jax.ShapeDtypeStruct((b, 1, d), jnp.float32),
        scratch_shapes=[pltpu.VMEM((e, d), jnp.float32)],
        compiler_params=pltpu.CompilerParams(
            dimension_semantics=("arbitrary",)),
    )(node_features_batch, src3, dst3, et3, W_rel, W_loop, row(b_gcn),
      Wq, row(bq), Wk, row(bk), Wv, row(bv), Wo, row(bo), W_mlp, row(b_mlp))
    return out.reshape(b, d)
